# Initial kernel scaffold; baseline (speedup 1.0000x reference)
import jax
import jax.numpy as jnp
from jax.experimental import pallas as pl


def kernel(dx, dy, dz_bucket, bias):
    raise NotImplementedError("write your pallas kernel here")


# speedup vs baseline: 52.4397x; 52.4397x over previous
import jax
import jax.numpy as jnp
from jax import lax
from jax.experimental import pallas as pl
from jax.experimental.pallas import tpu as pltpu
from jax.experimental.pallas import tpu_sc as plsc

NUM_HEADS = 12
TXY = 32
TZ = 32
L = 2048
N = L * L
HALF = TXY // 2
TABLE = HALF * HALF * TZ

NC = 2
NS = 16
NW = NC * NS
N_PER_W = N // NW
CHUNK = 1024
ROUNDS = N_PER_W // CHUNK
VECS = CHUNK // 16


def _rel_bias_kernel(dx_hbm, dy_hbm, dz_hbm, table_hbm, out_hbm,
                     table_v, dx_v, dy_v, dz_v, out_v):
    wid = lax.axis_index("s") * NC + lax.axis_index("c")
    base = wid * N_PER_W

    pltpu.sync_copy(table_hbm, table_v)

    def round_body(r, _):
        off = base + r * CHUNK
        pltpu.sync_copy(dx_hbm.at[pl.ds(off, CHUNK)], dx_v)
        pltpu.sync_copy(dy_hbm.at[pl.ds(off, CHUNK)], dy_v)
        pltpu.sync_copy(dz_hbm.at[pl.ds(off, CHUNK)], dz_v)

        def vec_body(i, _):
            s = pl.ds(i * 16, 16)
            flat = dx_v[s] * (HALF * TZ) + dy_v[s] * TZ + dz_v[s]
            for h in range(NUM_HEADS):
                out_v[h, s] = plsc.load_gather(table_v, [flat + h * TABLE])
            return 0

        lax.fori_loop(0, VECS, vec_body, 0)

        for h in range(NUM_HEADS):
            pltpu.sync_copy(out_v.at[h], out_hbm.at[h, pl.ds(off, CHUNK)])
        return 0

    lax.fori_loop(0, ROUNDS, round_body, 0)


@jax.jit
def kernel(dx, dy, dz_bucket, bias):
    dxf = dx.reshape(N)
    dyf = dy.reshape(N)
    dzf = dz_bucket.reshape(N)
    table = bias[:, HALF:, HALF:, :].reshape(NUM_HEADS * TABLE)

    mesh = plsc.VectorSubcoreMesh(core_axis_name="c", subcore_axis_name="s")
    out = pl.kernel(
        _rel_bias_kernel,
        mesh=mesh,
        compiler_params=pltpu.CompilerParams(needs_layout_passes=False),
        out_type=jax.ShapeDtypeStruct((NUM_HEADS, N), jnp.float32),
        scratch_types=[
            pltpu.VMEM((NUM_HEADS * TABLE,), jnp.float32),
            pltpu.VMEM((CHUNK,), jnp.int32),
            pltpu.VMEM((CHUNK,), jnp.int32),
            pltpu.VMEM((CHUNK,), jnp.int32),
            pltpu.VMEM((NUM_HEADS, CHUNK), jnp.float32),
        ],
    )(dxf, dyf, dzf, table)
    return out.reshape(NUM_HEADS, L, L)


# speedup vs baseline: 94.0482x; 1.7935x over previous
import jax
import jax.numpy as jnp
from jax import lax
from jax.experimental import pallas as pl
from jax.experimental.pallas import tpu as pltpu
from jax.experimental.pallas import tpu_sc as plsc

NUM_HEADS = 12
TXY = 32
TZ = 32
L = 2048
N = L * L
HALF = TXY // 2
TABLE = HALF * HALF * TZ

NC = 2
NS = 16
NW = NC * NS
N_PER_W = N // NW
CHUNK = 1024
ROUNDS = N_PER_W // CHUNK
VECS = CHUNK // 16
NBUF = 2


def _rel_bias_kernel(dx_hbm, dy_hbm, dz_hbm, table_hbm, out_hbm,
                     table_v, dx_v, dy_v, dz_v, out_v, in_sem, out_sem):
    wid = lax.axis_index("s") * NC + lax.axis_index("c")
    base = wid * N_PER_W

    def fire_in(r, b):
        off = base + r * CHUNK
        pltpu.async_copy(dx_hbm.at[pl.ds(off, CHUNK)], dx_v.at[b], in_sem.at[b])
        pltpu.async_copy(dy_hbm.at[pl.ds(off, CHUNK)], dy_v.at[b], in_sem.at[b])
        pltpu.async_copy(dz_hbm.at[pl.ds(off, CHUNK)], dz_v.at[b], in_sem.at[b])

    def drain_in(b):
        for ref in (dx_v, dy_v, dz_v):
            pltpu.make_async_copy(
                dx_hbm.at[pl.ds(0, CHUNK)], ref.at[b], in_sem.at[b]).wait()

    def fire_out(r, b):
        off = base + r * CHUNK
        pltpu.async_copy(
            out_v.at[b], out_hbm.at[:, pl.ds(off, CHUNK)], out_sem.at[b])

    def drain_out(b):
        pltpu.make_async_copy(
            out_v.at[b], out_hbm.at[:, pl.ds(0, CHUNK)], out_sem.at[b]).wait()

    pltpu.sync_copy(table_hbm, table_v)
    fire_in(0, 0)
    fire_in(1, 1)

    def group_body(g, _):
        for b in range(NBUF):
            r = 2 * g + b
            drain_in(b)

            @pl.when(g > 0)
            def _():
                drain_out(b)

            def vec_body(i, _):
                s = pl.ds(i * 16, 16)
                flat = dx_v[b, s] * (HALF * TZ) + dy_v[b, s] * TZ + dz_v[b, s]
                for h in range(NUM_HEADS):
                    out_v[b, h, s] = plsc.load_gather(
                        table_v, [flat + h * TABLE])
                return 0

            lax.fori_loop(0, VECS, vec_body, 0)
            fire_out(r, b)

            @pl.when(r + 2 < ROUNDS)
            def _():
                fire_in(r + 2, b)
        return 0

    lax.fori_loop(0, ROUNDS // NBUF, group_body, 0)
    drain_out(0)
    drain_out(1)


@jax.jit
def kernel(dx, dy, dz_bucket, bias):
    dxf = dx.reshape(N)
    dyf = dy.reshape(N)
    dzf = dz_bucket.reshape(N)
    table = bias[:, HALF:, HALF:, :].reshape(NUM_HEADS * TABLE)

    mesh = plsc.VectorSubcoreMesh(core_axis_name="c", subcore_axis_name="s")
    out = pl.kernel(
        _rel_bias_kernel,
        mesh=mesh,
        compiler_params=pltpu.CompilerParams(
            needs_layout_passes=False, use_tc_tiling_on_sc=False),
        out_type=jax.ShapeDtypeStruct((NUM_HEADS, N), jnp.float32),
        scratch_types=[
            pltpu.VMEM((NUM_HEADS * TABLE,), jnp.float32),
            pltpu.VMEM((NBUF, CHUNK), jnp.int32),
            pltpu.VMEM((NBUF, CHUNK), jnp.int32),
            pltpu.VMEM((NBUF, CHUNK), jnp.int32),
            pltpu.VMEM((NBUF, NUM_HEADS, CHUNK), jnp.float32),
            pltpu.SemaphoreType.DMA((NBUF,)),
            pltpu.SemaphoreType.DMA((NBUF,)),
        ],
    )(dxf, dyf, dzf, table)
    return out.reshape(NUM_HEADS, L, L)


# speedup vs baseline: 132.1565x; 1.4052x over previous
import jax
import jax.numpy as jnp
from jax import lax
from jax.experimental import pallas as pl
from jax.experimental.pallas import tpu as pltpu
from jax.experimental.pallas import tpu_sc as plsc

NUM_HEADS = 12
TXY = 32
TZ = 32
L = 2048
HALF = TXY // 2
TABLE = HALF * HALF * TZ

NC = 2
NS = 16
NW = NC * NS
ROWS_PER_W = L // NW
TR = 8
TC = 128
RB = ROWS_PER_W // TR
CB = L // TC
ROUNDS = RB * CB
NBUF = 2


def _rel_bias_kernel(dx_hbm, dy_hbm, dz_hbm, table_hbm, out_hbm,
                     table_v, dx_v, dy_v, dz_v, out_v, in_sem, out_sem):
    wid = lax.axis_index("s") * NC + lax.axis_index("c")
    row_base = wid * ROWS_PER_W

    def tile_of(r):
        row0 = row_base + (r // CB) * TR
        col0 = (r % CB) * TC
        return row0, col0

    def fire_in(r, b):
        row0, col0 = tile_of(r)
        for src, dst in ((dx_hbm, dx_v), (dy_hbm, dy_v), (dz_hbm, dz_v)):
            pltpu.async_copy(
                src.at[pl.ds(row0, TR), pl.ds(col0, TC)], dst.at[b],
                in_sem.at[b])

    def drain_in(b):
        for ref in (dx_v, dy_v, dz_v):
            pltpu.make_async_copy(
                dx_hbm.at[pl.ds(0, TR), pl.ds(0, TC)], ref.at[b],
                in_sem.at[b]).wait()

    def fire_out(r, b):
        row0, col0 = tile_of(r)
        for h in range(NUM_HEADS):
            pltpu.async_copy(
                out_v.at[b, h],
                out_hbm.at[h, pl.ds(row0, TR), pl.ds(col0, TC)],
                out_sem.at[b])

    def drain_out(b):
        for h in range(NUM_HEADS):
            pltpu.make_async_copy(
                out_v.at[b, h], out_hbm.at[h, pl.ds(0, TR), pl.ds(0, TC)],
                out_sem.at[b]).wait()

    pltpu.sync_copy(table_hbm, table_v)
    fire_in(0, 0)
    fire_in(1, 1)

    def group_body(g, _):
        for b in range(NBUF):
            r = NBUF * g + b
            drain_in(b)

            @pl.when(g > 0)
            def _():
                drain_out(b)

            def row_body(rr, _):
                for i in range(TC // 16):
                    s = pl.ds(i * 16, 16)
                    flat = (dx_v[b, rr, s] * (HALF * TZ)
                            + dy_v[b, rr, s] * TZ + dz_v[b, rr, s])
                    for h in range(NUM_HEADS):
                        out_v[b, h, rr, s] = plsc.load_gather(
                            table_v, [flat + h * TABLE])
                return 0

            lax.fori_loop(0, TR, row_body, 0)
            fire_out(r, b)

            @pl.when(r + NBUF < ROUNDS)
            def _():
                fire_in(r + NBUF, b)
        return 0

    lax.fori_loop(0, ROUNDS // NBUF, group_body, 0)
    drain_out(0)
    drain_out(1)


@jax.jit
def kernel(dx, dy, dz_bucket, bias):
    table = bias[:, HALF:, HALF:, :].reshape(NUM_HEADS * TABLE)

    mesh = plsc.VectorSubcoreMesh(core_axis_name="c", subcore_axis_name="s")
    return pl.kernel(
        _rel_bias_kernel,
        mesh=mesh,
        compiler_params=pltpu.CompilerParams(
            needs_layout_passes=False, use_tc_tiling_on_sc=True),
        out_type=jax.ShapeDtypeStruct((NUM_HEADS, L, L), jnp.float32),
        scratch_types=[
            pltpu.VMEM((NUM_HEADS * TABLE,), jnp.float32),
            pltpu.VMEM((NBUF, TR, TC), jnp.int32),
            pltpu.VMEM((NBUF, TR, TC), jnp.int32),
            pltpu.VMEM((NBUF, TR, TC), jnp.int32),
            pltpu.VMEM((NBUF, NUM_HEADS, TR, TC), jnp.float32),
            pltpu.SemaphoreType.DMA((NBUF,)),
            pltpu.SemaphoreType.DMA((NBUF,)),
        ],
    )(dx, dy, dz_bucket, table)
